# baseline (speedup 1.0000x reference)
import functools

import jax
import jax.numpy as jnp
from jax import lax
from jax.experimental import pallas as pl
from jax.experimental.pallas import tpu as pltpu
from jax.experimental.pallas import tpu_sc as plsc

D_MODEL = 2048
BATCH = 16384

_info = plsc.get_sparse_core_info()
_NC, _NS = _info.num_cores, _info.num_subcores
_NW = _NC * _NS
_BPW = BATCH // _NW
_GRP = 16
_NGRP = _BPW // _GRP


@functools.partial(
    pl.kernel,
    mesh=plsc.VectorSubcoreMesh(core_axis_name="c", subcore_axis_name="s"),
    out_type=jax.ShapeDtypeStruct((BATCH, D_MODEL), jnp.float32),
    scratch_types=[
        pltpu.VMEM((_BPW,), jnp.int32),
        pltpu.VMEM((5, D_MODEL), jnp.float32),
        pltpu.SemaphoreType.DMA,
        pltpu.SemaphoreType.DMA,
    ],
)
def _sc_embed(idx_hbm, table_hbm, out_hbm, idx_s, table_v, sem, stage_sem):
    wid = lax.axis_index("s") * _NC + lax.axis_index("c")
    base = wid * _BPW
    pltpu.async_copy(table_hbm, table_v, stage_sem)
    idx_cp = pltpu.async_copy(idx_hbm.at[pl.ds(base, _BPW)], idx_s, stage_sem)
    pltpu.make_async_copy(table_hbm, table_v, stage_sem).wait()
    idx_cp.wait()

    def issue_group(g):
        idx16 = idx_s[pl.ds(g * _GRP, _GRP)]
        for r in range(_GRP):
            s = idx16[r]
            pltpu.async_copy(
                table_v.at[pl.ds(s, 1)],
                out_hbm.at[pl.ds(base + g * _GRP + r, 1)],
                sem,
            )

    def drain_one_group():
        pltpu.make_async_copy(
            out_hbm.at[pl.ds(0, _GRP)], out_hbm.at[pl.ds(base, _GRP)], sem
        ).wait()

    issue_group(0)
    issue_group(1)

    @pl.loop(2, _NGRP)
    def _(g):
        issue_group(g)
        drain_one_group()

    drain_one_group()
    drain_one_group()


def kernel(scene_id, embedding_weight):
    if scene_id.ndim > 1:
        scene_id = jnp.squeeze(scene_id, axis=-1)
    idx = scene_id.astype(jnp.int32)
    return _sc_embed(idx, embedding_weight)


# speedup vs baseline: 1.0248x; 1.0248x over previous
import functools

import jax
import jax.numpy as jnp
from jax import lax
from jax.experimental import pallas as pl
from jax.experimental.pallas import tpu as pltpu
from jax.experimental.pallas import tpu_sc as plsc

D_MODEL = 2048
BATCH = 16384

_info = plsc.get_sparse_core_info()
_NC, _NS = _info.num_cores, _info.num_subcores
_NW = _NC * _NS
_BPW = BATCH // _NW
_GRP = 16
_NGRP = _BPW // _GRP


@functools.partial(
    pl.kernel,
    mesh=plsc.VectorSubcoreMesh(core_axis_name="c", subcore_axis_name="s"),
    out_type=jax.ShapeDtypeStruct((BATCH, D_MODEL), jnp.float32),
    scratch_types=[
        pltpu.VMEM((_BPW,), jnp.int32),
        pltpu.VMEM((5, D_MODEL), jnp.float32),
        pltpu.SemaphoreType.DMA,
        pltpu.SemaphoreType.DMA,
    ],
)
def _sc_embed(idx_hbm, table_hbm, out_hbm, idx_s, table_v, sem, stage_sem):
    wid = lax.axis_index("s") * _NC + lax.axis_index("c")
    base = wid * _BPW
    pltpu.sync_copy(table_hbm, table_v)
    pltpu.sync_copy(idx_hbm.at[pl.ds(base, _BPW)], idx_s)

    def issue_group(g):
        idx16 = idx_s[pl.ds(g * _GRP, _GRP)]
        for r in range(_GRP):
            s = idx16[r]
            pltpu.async_copy(
                table_v.at[pl.ds(s, 1)],
                out_hbm.at[pl.ds(base + g * _GRP + r, 1)],
                sem,
            )

    def drain_one_group():
        pltpu.make_async_copy(
            out_hbm.at[pl.ds(0, _GRP)], out_hbm.at[pl.ds(base, _GRP)], sem
        ).wait()

    issue_group(0)

    @pl.loop(1, _NGRP)
    def _(g):
        issue_group(g)
        drain_one_group()

    drain_one_group()


def kernel(scene_id, embedding_weight):
    if scene_id.ndim > 1:
        scene_id = jnp.squeeze(scene_id, axis=-1)
    idx = scene_id.astype(jnp.int32)
    return _sc_embed(idx, embedding_weight)


# speedup vs baseline: 1.0339x; 1.0088x over previous
import functools

import jax
import jax.numpy as jnp
from jax import lax
from jax.experimental import pallas as pl
from jax.experimental.pallas import tpu as pltpu
from jax.experimental.pallas import tpu_sc as plsc

D_MODEL = 2048
BATCH = 16384

_info = plsc.get_sparse_core_info()
_NC, _NS = _info.num_cores, _info.num_subcores
_NW = _NC * _NS
_BPW = BATCH // _NW
_GRP = 8
_NGRP = _BPW // _GRP


@functools.partial(
    pl.kernel,
    mesh=plsc.VectorSubcoreMesh(core_axis_name="c", subcore_axis_name="s"),
    out_type=jax.ShapeDtypeStruct((BATCH, D_MODEL), jnp.float32),
    scratch_types=[
        pltpu.VMEM((_BPW,), jnp.int32),
        pltpu.VMEM((5, D_MODEL), jnp.float32),
        pltpu.SemaphoreType.DMA,
        pltpu.SemaphoreType.DMA,
    ],
)
def _sc_embed(idx_hbm, table_hbm, out_hbm, idx_s, table_v, sem, stage_sem):
    wid = lax.axis_index("s") * _NC + lax.axis_index("c")
    base = wid * _BPW
    pltpu.sync_copy(table_hbm, table_v)
    pltpu.sync_copy(idx_hbm.at[pl.ds(base, _BPW)], idx_s)

    def issue_group(g):
        idx16 = idx_s[pl.ds(g * _GRP, _GRP)]
        for r in range(_GRP):
            s = idx16[r]
            pltpu.async_copy(
                table_v.at[pl.ds(s, 1)],
                out_hbm.at[pl.ds(base + g * _GRP + r, 1)],
                sem,
            )

    def drain_one_group():
        pltpu.make_async_copy(
            out_hbm.at[pl.ds(0, _GRP)], out_hbm.at[pl.ds(base, _GRP)], sem
        ).wait()

    issue_group(0)

    @pl.loop(1, _NGRP)
    def _(g):
        issue_group(g)
        drain_one_group()

    drain_one_group()


def kernel(scene_id, embedding_weight):
    if scene_id.ndim > 1:
        scene_id = jnp.squeeze(scene_id, axis=-1)
    idx = scene_id.astype(jnp.int32)
    return _sc_embed(idx, embedding_weight)


# speedup vs baseline: 1.0533x; 1.0188x over previous
import functools

import jax
import jax.numpy as jnp
from jax import lax
from jax.experimental import pallas as pl
from jax.experimental.pallas import tpu as pltpu
from jax.experimental.pallas import tpu_sc as plsc

D_MODEL = 2048
BATCH = 16384

_info = plsc.get_sparse_core_info()
_NC, _NS = _info.num_cores, _info.num_subcores
_NW = _NC * _NS
_BPW = BATCH // _NW
_GRP = 4
_NGRP = _BPW // _GRP


@functools.partial(
    pl.kernel,
    mesh=plsc.VectorSubcoreMesh(core_axis_name="c", subcore_axis_name="s"),
    out_type=jax.ShapeDtypeStruct((BATCH, D_MODEL), jnp.float32),
    scratch_types=[
        pltpu.VMEM((_BPW,), jnp.int32),
        pltpu.VMEM((5, D_MODEL), jnp.float32),
        pltpu.SemaphoreType.DMA,
        pltpu.SemaphoreType.DMA,
    ],
)
def _sc_embed(idx_hbm, table_hbm, out_hbm, idx_s, table_v, sem, stage_sem):
    wid = lax.axis_index("s") * _NC + lax.axis_index("c")
    base = wid * _BPW
    pltpu.sync_copy(table_hbm, table_v)
    pltpu.sync_copy(idx_hbm.at[pl.ds(base, _BPW)], idx_s)

    def issue_group(g):
        idx16 = idx_s[pl.ds(g * _GRP, _GRP)]
        for r in range(_GRP):
            s = idx16[r]
            pltpu.async_copy(
                table_v.at[pl.ds(s, 1)],
                out_hbm.at[pl.ds(base + g * _GRP + r, 1)],
                sem,
            )

    def drain_one_group():
        pltpu.make_async_copy(
            out_hbm.at[pl.ds(0, _GRP)], out_hbm.at[pl.ds(base, _GRP)], sem
        ).wait()

    issue_group(0)

    @pl.loop(1, _NGRP)
    def _(g):
        issue_group(g)
        drain_one_group()

    drain_one_group()


def kernel(scene_id, embedding_weight):
    if scene_id.ndim > 1:
        scene_id = jnp.squeeze(scene_id, axis=-1)
    idx = scene_id.astype(jnp.int32)
    return _sc_embed(idx, embedding_weight)
